# baseline (speedup 1.0000x reference)
import functools

import jax
import jax.numpy as jnp
from jax import lax
from jax.experimental import pallas as pl
from jax.experimental.pallas import tpu as pltpu
from jax.experimental.pallas import tpu_sc as plsc

N = 10000
E = 320000
NC = 2
NS = 16
NPAD = 10240
RPT = NPAD // NS
EPT = E // NS
CH = 80
NCH = EPT // CH

F32 = jnp.float32
BM = 2048
GRID = NPAD // BM


def _sc_mesh():
    return plsc.VectorSubcoreMesh(core_axis_name="c", subcore_axis_name="s")


@functools.partial(
    pl.kernel,
    mesh=_sc_mesh(),
    out_type=jax.ShapeDtypeStruct((NC, NPAD, 128), F32),
    scratch_types=[
        pltpu.VMEM((CH,), jnp.int32),
        pltpu.VMEM((CH,), jnp.int32),
        pltpu.VMEM((CH, 128), F32),
        pltpu.VMEM_SHARED((NPAD, 128), F32),
        pltpu.SemaphoreType.DMA,
        pltpu.SemaphoreType.DMA,
    ],
)
def _deg_kernel(edges, deg_out, idx_a, idx_b, ones_v, acc_s, sem_a, sem_b):
    c = lax.axis_index("c")
    t = lax.axis_index("s")
    idx = (idx_a, idx_b)
    sem = (sem_a, sem_b)

    def fill(val):
        def row(i, carry):
            def lane(j, carry2):
                ones_v[i, pl.ds(j * 16, 16)] = jnp.full((16,), val, F32)
                return carry2

            lax.fori_loop(0, 128 // 16, lane, 0)
            return carry

        lax.fori_loop(0, CH, row, 0)

    fill(0.0)

    def zero_acc(k, carry):
        pltpu.sync_copy(ones_v, acc_s.at[pl.ds(t * RPT + k * CH, CH)])
        return carry

    lax.fori_loop(0, RPT // CH, zero_acc, 0)
    fill(1.0)
    plsc.subcore_barrier()

    def start(ci, b):
        base = c * E + t * EPT + ci * CH
        pltpu.async_copy(edges.at[pl.ds(base, CH)], idx[b], sem[b])

    def finish(b):
        pltpu.make_async_copy(edges.at[pl.ds(0, CH)], idx[b], sem[b]).wait()

    def scatter(b):
        pltpu.sync_copy(ones_v, acc_s.at[idx[b]], add=True)

    start(0, 0)

    def pair(i, carry):
        start(2 * i + 1, 1)
        finish(0)
        scatter(0)
        start((2 * i + 2) % NCH, 0)
        finish(1)
        scatter(1)
        return carry

    lax.fori_loop(0, NCH // 2, pair, 0)
    finish(0)
    plsc.subcore_barrier()

    def out_cp(k, carry):
        r0 = t * RPT + k * CH
        pltpu.sync_copy(acc_s.at[pl.ds(r0, CH)], ones_v)
        pltpu.sync_copy(ones_v, deg_out.at[c, pl.ds(r0, CH)])
        return carry

    lax.fori_loop(0, RPT // CH, out_cp, 0)


def _make_agg(H, edge_split=False):
    ept = E // (NC * NS) if edge_split else EPT
    nch = ept // CH

    @functools.partial(
        pl.kernel,
        mesh=_sc_mesh(),
        out_type=jax.ShapeDtypeStruct((NC, NPAD, H), F32),
        scratch_types=[
            pltpu.VMEM((2, CH), jnp.int32),
            pltpu.VMEM((2, CH), jnp.int32),
            pltpu.VMEM((CH,), jnp.int32),
            pltpu.VMEM((CH,), jnp.int32),
            pltpu.VMEM((CH, H), F32),
            pltpu.VMEM((CH, H), F32),
            pltpu.VMEM_SHARED((NPAD, H), F32),
            pltpu.SemaphoreType.DMA,
            pltpu.SemaphoreType.DMA,
            pltpu.SemaphoreType.DMA,
            pltpu.SemaphoreType.DMA,
            pltpu.SemaphoreType.DMA,
            pltpu.SemaphoreType.DMA,
        ],
    )
    def agg_kernel(edges4, feats, out, idx_a, idx_b, six_a, six_b, rows_a,
                   rows_b, acc_s, gsem_a, gsem_b, isem_a, isem_b, ssem_a,
                   ssem_b):
        c = lax.axis_index("c")
        t = lax.axis_index("s")
        idx = (idx_a, idx_b)
        six = (six_a, six_b)
        rows = (rows_a, rows_b)
        gsem = (gsem_a, gsem_b)
        isem = (isem_a, isem_b)
        ssem = (ssem_a, ssem_b)

        def zero_rows(i, carry):
            def zl(j, carry2):
                rows_a[i, pl.ds(j * 16, 16)] = jnp.zeros((16,), F32)
                rows_b[i, pl.ds(j * 16, 16)] = jnp.zeros((16,), F32)
                return carry2

            lax.fori_loop(0, H // 16, zl, 0)
            return carry

        lax.fori_loop(0, CH, zero_rows, 0)
        for j in range(CH // 16):
            six_b[pl.ds(j * 16, 16)] = jnp.zeros((16,), jnp.int32)

        def zero_acc(k, carry):
            pltpu.sync_copy(rows_a, acc_s.at[pl.ds(t * RPT + k * CH, CH)])
            return carry

        lax.fori_loop(0, RPT // CH, zero_acc, 0)
        plsc.subcore_barrier()

        def idx_src(ci):
            if edge_split:
                return edges4.at[c, t, ci]
            return edges4.at[t, ci]

        def load_idx(ci, b):
            pltpu.async_copy(idx_src(ci), idx[b], isem[b])

        def wait_idx(b):
            pltpu.make_async_copy(idx_src(0), idx[b], isem[b]).wait()

        def start_gather(b):
            wait_idx(b)
            if not edge_split:
                for j in range(CH // 16):
                    s = idx[b][0, pl.ds(j * 16, 16)]
                    idx[b][0, pl.ds(j * 16, 16)] = s * 2 + c
            pltpu.async_copy(feats.at[idx[b].at[0]], rows[b], gsem[b])

        def finish_gather(b):
            pltpu.make_async_copy(feats.at[idx[b].at[0]], rows[b],
                                  gsem[b]).wait()

        def copy_six(b):
            for j in range(CH // 16):
                six[b][pl.ds(j * 16, 16)] = idx[b][1, pl.ds(j * 16, 16)]

        def start_scatter(b):
            pltpu.async_copy(rows[b], acc_s.at[six[b]], ssem[b], add=True)

        def wait_scatter(b):
            pltpu.make_async_copy(rows[b], acc_s.at[six[b]], ssem[b]).wait()

        load_idx(0, 0)
        load_idx(1, 1)
        start_gather(0)
        start_scatter(1)

        def pair(i, carry):
            wait_scatter(1)
            start_gather(1)
            finish_gather(0)
            copy_six(0)
            start_scatter(0)
            load_idx((2 * i + 2) % nch, 0)
            finish_gather(1)
            copy_six(1)
            start_scatter(1)
            load_idx((2 * i + 3) % nch, 1)
            wait_scatter(0)
            start_gather(0)
            return carry

        lax.fori_loop(0, nch // 2, pair, 0)
        finish_gather(0)
        if nch % 2 == 1:
            copy_six(0)
            start_scatter(0)
            wait_scatter(0)
        wait_scatter(1)
        wait_idx(1)
        plsc.subcore_barrier()

        def out_cp(k, carry):
            r0 = t * RPT + k * CH
            pltpu.sync_copy(acc_s.at[pl.ds(r0, CH)], rows_a)
            pltpu.sync_copy(rows_a, out.at[c, pl.ds(r0, CH)])
            return carry

        lax.fori_loop(0, RPT // CH, out_cp, 0)

    return agg_kernel


_agg128 = _make_agg(128)

_agg_last = _make_agg(128, edge_split=True)


def _dot(a, b):
    return lax.dot_general(
        a, b, (((1,), (0,)), ((), ())),
        precision=lax.Precision.HIGHEST,
        preferred_element_type=F32,
    )


def _silu(v):
    return v / (1.0 + jnp.exp(-v))


def _norms(deg_blk):
    d0 = deg_blk[:, 0:1]
    d1 = deg_blk[:, 1:2]
    ns = jnp.where(d0 > 0, lax.rsqrt(jnp.maximum(d0, 1.0)), 0.0)
    nd = jnp.where(d1 > 0, lax.rsqrt(jnp.maximum(d1, 1.0)), 0.0)
    return ns, nd


def _rows(i):
    return (i, 0)


def _full(i):
    return (0, 0)


def _stage_a(xp, deg_t):
    def body(x_ref, deg_ref, o_ref):
        ns, _ = _norms(deg_ref[...])
        o_ref[...] = x_ref[...] * ns

    return pl.pallas_call(
        body,
        grid=(GRID,),
        in_specs=[
            pl.BlockSpec((BM, 128), _rows),
            pl.BlockSpec((BM, 2), _rows),
        ],
        out_specs=pl.BlockSpec((BM, 128), _rows),
        out_shape=jax.ShapeDtypeStruct((NPAD, 128), F32),
    )(xp, deg_t)


def _stage_b(a0, a1, deg_t, b1, w1, wh1):
    def body(a0_ref, a1_ref, deg_ref, b_ref, w1_ref, w_ref, o_ref):
        ns, nd = _norms(deg_ref[...])
        y = _dot(a0_ref[...] + a1_ref[...], w1_ref[...])
        t = (y * nd + b_ref[...]) * ns
        o_ref[...] = _dot(t, w_ref[...])

    return pl.pallas_call(
        body,
        grid=(GRID,),
        in_specs=[
            pl.BlockSpec((BM, 128), _rows),
            pl.BlockSpec((BM, 128), _rows),
            pl.BlockSpec((BM, 2), _rows),
            pl.BlockSpec((1, 256), _full),
            pl.BlockSpec((128, 256), _full),
            pl.BlockSpec((256, 256), _full),
        ],
        out_specs=pl.BlockSpec((BM, 256), _rows),
        out_shape=jax.ShapeDtypeStruct((NPAD, 256), F32),
    )(a0, a1, deg_t, b1, w1, wh1)


def _stage_c(a0, a1, deg_t, bh1, wm1, bm1, wm2, bm2, wh2):
    def body(a0_ref, a1_ref, deg_ref, bh_ref, wm1_ref, bm1_ref, wm2_ref,
             bm2_ref, wh2_ref, o_ref):
        ns, nd = _norms(deg_ref[...])
        bh = bh_ref[...]
        h0 = _silu(a0_ref[...] * nd + bh[:, :128])
        h1 = _silu(a1_ref[...] * nd + bh[:, 128:])
        z = _dot(_silu(h0), wm1_ref[:128, :]) + _dot(_silu(h1), wm1_ref[128:, :])
        z = z + bm1_ref[...]
        hm = _dot(_silu(z), wm2_ref[...]) + bm2_ref[...]
        o_ref[...] = _dot(hm * ns, wh2_ref[...])

    return pl.pallas_call(
        body,
        grid=(GRID,),
        in_specs=[
            pl.BlockSpec((BM, 128), _rows),
            pl.BlockSpec((BM, 128), _rows),
            pl.BlockSpec((BM, 2), _rows),
            pl.BlockSpec((1, 256), _full),
            pl.BlockSpec((256, 64), _full),
            pl.BlockSpec((1, 64), _full),
            pl.BlockSpec((64, 256), _full),
            pl.BlockSpec((1, 256), _full),
            pl.BlockSpec((256, 256), _full),
        ],
        out_specs=pl.BlockSpec((BM, 256), _rows),
        out_shape=jax.ShapeDtypeStruct((NPAD, 256), F32),
    )(a0, a1, deg_t, bh1, wm1, bm1, wm2, bm2, wh2)


def _stage_d(a0, a1, deg_t, bh2, w2):
    def body(a0_ref, a1_ref, deg_ref, b_ref, w_ref, o_ref):
        ns, nd = _norms(deg_ref[...])
        b = b_ref[...]
        t0 = _silu(a0_ref[...] * nd + b[:, :128]) * ns
        t1 = _silu(a1_ref[...] * nd + b[:, 128:]) * ns
        o_ref[...] = _dot(t0, w_ref[:128, :]) + _dot(t1, w_ref[128:, :])

    return pl.pallas_call(
        body,
        grid=(GRID,),
        in_specs=[
            pl.BlockSpec((BM, 128), _rows),
            pl.BlockSpec((BM, 128), _rows),
            pl.BlockSpec((BM, 2), _rows),
            pl.BlockSpec((1, 256), _full),
            pl.BlockSpec((256, 128), _full),
        ],
        out_specs=pl.BlockSpec((BM, 128), _rows),
        out_shape=jax.ShapeDtypeStruct((NPAD, 128), F32),
    )(a0, a1, deg_t, bh2, w2)


def _stage_e(a0, a1, deg_t, b2):
    def body(a0_ref, a1_ref, deg_ref, b_ref, o_ref):
        _, nd = _norms(deg_ref[...])
        o_ref[...] = (a0_ref[...] + a1_ref[...]) * nd + b_ref[...]

    return pl.pallas_call(
        body,
        grid=(GRID,),
        in_specs=[
            pl.BlockSpec((BM, 128), _rows),
            pl.BlockSpec((BM, 128), _rows),
            pl.BlockSpec((BM, 2), _rows),
            pl.BlockSpec((1, 128), _full),
        ],
        out_specs=pl.BlockSpec((BM, 128), _rows),
        out_shape=jax.ShapeDtypeStruct((NPAD, 128), F32),
    )(a0, a1, deg_t, b2)


def _split_rows(h):
    n, d = h.shape
    return h.reshape(n, 2, d // 2).reshape(2 * n, d // 2)


def kernel(x, edge_index, W1, b1, Wh1, bh1, Wh2, bh2, Wm1, bm1, Wm2, bm2,
           W2, b2):
    ei = edge_index.astype(jnp.int32)
    eif = ei.reshape(2 * E)
    ein = ei.reshape(2, NS, EPT // CH, CH).transpose(1, 2, 0, 3)
    einl = ei.reshape(2, NC, NS, E // (NC * NS) // CH, CH)
    einl = einl.transpose(1, 2, 3, 0, 4)
    deg_raw = _deg_kernel(eif)
    deg_t = deg_raw[:, :, 0].T

    xp = jnp.pad(x, ((0, NPAD - N), (0, 0)))
    b1r = b1.reshape(1, -1)
    bh1r = bh1.reshape(1, -1)
    bh2r = bh2.reshape(1, -1)
    bm1r = bm1.reshape(1, -1)
    bm2r = bm2.reshape(1, -1)
    b2r = b2.reshape(1, -1)

    xs = _stage_a(xp, deg_t)
    a1 = _agg_last(einl, xs)
    h2 = _stage_b(a1[0], a1[1], deg_t, b1r, W1, Wh1)
    a2 = _agg128(ein, _split_rows(h2))
    h3 = _stage_c(a2[0], a2[1], deg_t, bh1r, Wm1, bm1r, Wm2, bm2r, Wh2)
    a3 = _agg128(ein, _split_rows(h3))
    h4 = _stage_d(a3[0], a3[1], deg_t, bh2r, W2)
    a4 = _agg_last(einl, h4)
    out = _stage_e(a4[0], a4[1], deg_t, b2r)
    return out[:N, :]


# speedup vs baseline: 1.0657x; 1.0657x over previous
import functools

import jax
import jax.numpy as jnp
from jax import lax
from jax.experimental import pallas as pl
from jax.experimental.pallas import tpu as pltpu
from jax.experimental.pallas import tpu_sc as plsc

N = 10000
E = 320000
NC = 2
NS = 16
NPAD = 10240
RPT = NPAD // NS
EPT = E // NS
CH = 80
NCH = EPT // CH

F32 = jnp.float32
BM = 2048
GRID = NPAD // BM


def _sc_mesh():
    return plsc.VectorSubcoreMesh(core_axis_name="c", subcore_axis_name="s")


@functools.partial(
    pl.kernel,
    mesh=_sc_mesh(),
    out_type=jax.ShapeDtypeStruct((NC, NPAD, 128), F32),
    scratch_types=[
        pltpu.VMEM((CH,), jnp.int32),
        pltpu.VMEM((CH,), jnp.int32),
        pltpu.VMEM((CH, 128), F32),
        pltpu.VMEM_SHARED((NPAD, 128), F32),
        pltpu.SemaphoreType.DMA,
        pltpu.SemaphoreType.DMA,
    ],
)
def _deg_kernel(edges, deg_out, idx_a, idx_b, ones_v, acc_s, sem_a, sem_b):
    c = lax.axis_index("c")
    t = lax.axis_index("s")
    idx = (idx_a, idx_b)
    sem = (sem_a, sem_b)

    def fill(val):
        def row(i, carry):
            def lane(j, carry2):
                ones_v[i, pl.ds(j * 16, 16)] = jnp.full((16,), val, F32)
                return carry2

            lax.fori_loop(0, 128 // 16, lane, 0)
            return carry

        lax.fori_loop(0, CH, row, 0)

    fill(0.0)

    def zero_acc(k, carry):
        pltpu.sync_copy(ones_v, acc_s.at[pl.ds(t * RPT + k * CH, CH)])
        return carry

    lax.fori_loop(0, RPT // CH, zero_acc, 0)
    fill(1.0)
    plsc.subcore_barrier()

    def start(ci, b):
        base = c * E + t * EPT + ci * CH
        pltpu.async_copy(edges.at[pl.ds(base, CH)], idx[b], sem[b])

    def finish(b):
        pltpu.make_async_copy(edges.at[pl.ds(0, CH)], idx[b], sem[b]).wait()

    def scatter(b):
        pltpu.sync_copy(ones_v, acc_s.at[idx[b]], add=True)

    start(0, 0)

    def pair(i, carry):
        start(2 * i + 1, 1)
        finish(0)
        scatter(0)
        start((2 * i + 2) % NCH, 0)
        finish(1)
        scatter(1)
        return carry

    lax.fori_loop(0, NCH // 2, pair, 0)
    finish(0)
    plsc.subcore_barrier()

    def out_cp(k, carry):
        r0 = t * RPT + k * CH
        pltpu.sync_copy(acc_s.at[pl.ds(r0, CH)], ones_v)
        pltpu.sync_copy(ones_v, deg_out.at[c, pl.ds(r0, CH)])
        return carry

    lax.fori_loop(0, RPT // CH, out_cp, 0)


def _make_agg(H, edge_split=False):
    ept = E // (NC * NS) if edge_split else EPT
    nch = ept // CH

    @functools.partial(
        pl.kernel,
        mesh=_sc_mesh(),
        out_type=jax.ShapeDtypeStruct((NC, NPAD, H), F32),
        scratch_types=[
            pltpu.VMEM((2, CH), jnp.int32),
            pltpu.VMEM((2, CH), jnp.int32),
            pltpu.VMEM((CH, H), F32),
            pltpu.VMEM((CH, H), F32),
            pltpu.VMEM_SHARED((NPAD, H), F32),
            pltpu.SemaphoreType.DMA,
            pltpu.SemaphoreType.DMA,
            pltpu.SemaphoreType.DMA,
            pltpu.SemaphoreType.DMA,
        ],
    )
    def agg_kernel(edges4, feats, out, idx_a, idx_b, rows_a, rows_b, acc_s,
                   sem_a, sem_b, isem_a, isem_b):
        c = lax.axis_index("c")
        t = lax.axis_index("s")
        idx = (idx_a, idx_b)
        rows = (rows_a, rows_b)
        sem = (sem_a, sem_b)
        isem = (isem_a, isem_b)

        def zero_rows(i, carry):
            def zl(j, carry2):
                rows_a[i, pl.ds(j * 16, 16)] = jnp.zeros((16,), F32)
                return carry2

            lax.fori_loop(0, H // 16, zl, 0)
            return carry

        lax.fori_loop(0, CH, zero_rows, 0)

        def zero_acc(k, carry):
            pltpu.sync_copy(rows_a, acc_s.at[pl.ds(t * RPT + k * CH, CH)])
            return carry

        lax.fori_loop(0, RPT // CH, zero_acc, 0)
        plsc.subcore_barrier()

        def idx_src(ci):
            if edge_split:
                return edges4.at[c, t, ci]
            return edges4.at[t, ci]

        def load_idx(ci, b):
            pltpu.async_copy(idx_src(ci), idx[b], isem[b])

        def wait_idx(b):
            pltpu.make_async_copy(idx_src(0), idx[b], isem[b]).wait()

        def launch_gather(b):
            wait_idx(b)
            if not edge_split:
                for j in range(CH // 16):
                    s = idx[b][0, pl.ds(j * 16, 16)]
                    idx[b][0, pl.ds(j * 16, 16)] = s * 2 + c
            pltpu.async_copy(feats.at[idx[b].at[0]], rows[b], sem[b])

        def finish(b):
            pltpu.make_async_copy(feats.at[idx[b].at[0]], rows[b],
                                  sem[b]).wait()

        def scatter(b):
            pltpu.sync_copy(rows[b], acc_s.at[idx[b].at[1]], add=True)

        load_idx(0, 0)
        load_idx(1, 1)
        launch_gather(0)

        def pair(i, carry):
            launch_gather(1)
            finish(0)
            scatter(0)
            load_idx((2 * i + 2) % nch, 0)
            finish(1)
            scatter(1)
            launch_gather(0)
            load_idx((2 * i + 3) % nch, 1)
            return carry

        lax.fori_loop(0, nch // 2, pair, 0)
        finish(0)
        if nch % 2 == 1:
            scatter(0)
        wait_idx(1)
        plsc.subcore_barrier()

        def out_cp(k, carry):
            r0 = t * RPT + k * CH
            pltpu.sync_copy(acc_s.at[pl.ds(r0, CH)], rows_a)
            pltpu.sync_copy(rows_a, out.at[c, pl.ds(r0, CH)])
            return carry

        lax.fori_loop(0, RPT // CH, out_cp, 0)

    return agg_kernel


_agg128 = _make_agg(128)

_agg_last = _make_agg(128, edge_split=True)


def _dot(a, b):
    return lax.dot_general(
        a, b, (((1,), (0,)), ((), ())),
        preferred_element_type=F32,
    )


def _silu(v):
    return v / (1.0 + jnp.exp(-v))


def _norms(deg_blk):
    d0 = deg_blk[:, 0:1]
    d1 = deg_blk[:, 1:2]
    ns = jnp.where(d0 > 0, lax.rsqrt(jnp.maximum(d0, 1.0)), 0.0)
    nd = jnp.where(d1 > 0, lax.rsqrt(jnp.maximum(d1, 1.0)), 0.0)
    return ns, nd


def _rows(i):
    return (i, 0)


def _full(i):
    return (0, 0)


def _stage_a(xp, deg_t):
    def body(x_ref, deg_ref, o_ref):
        ns, _ = _norms(deg_ref[...])
        o_ref[...] = x_ref[...] * ns

    return pl.pallas_call(
        body,
        grid=(GRID,),
        in_specs=[
            pl.BlockSpec((BM, 128), _rows),
            pl.BlockSpec((BM, 2), _rows),
        ],
        out_specs=pl.BlockSpec((BM, 128), _rows),
        out_shape=jax.ShapeDtypeStruct((NPAD, 128), F32),
    )(xp, deg_t)


def _stage_b(a0, a1, deg_t, b1, w1, wh1):
    def body(a0_ref, a1_ref, deg_ref, b_ref, w1_ref, w_ref, o_ref):
        ns, nd = _norms(deg_ref[...])
        y = _dot(a0_ref[...] + a1_ref[...], w1_ref[...])
        t = (y * nd + b_ref[...]) * ns
        o_ref[...] = _dot(t, w_ref[...])

    return pl.pallas_call(
        body,
        grid=(GRID,),
        in_specs=[
            pl.BlockSpec((BM, 128), _rows),
            pl.BlockSpec((BM, 128), _rows),
            pl.BlockSpec((BM, 2), _rows),
            pl.BlockSpec((1, 256), _full),
            pl.BlockSpec((128, 256), _full),
            pl.BlockSpec((256, 256), _full),
        ],
        out_specs=pl.BlockSpec((BM, 256), _rows),
        out_shape=jax.ShapeDtypeStruct((NPAD, 256), F32),
    )(a0, a1, deg_t, b1, w1, wh1)


def _stage_c(a0, a1, deg_t, bh1, wm1, bm1, wm2, bm2, wh2):
    def body(a0_ref, a1_ref, deg_ref, bh_ref, wm1_ref, bm1_ref, wm2_ref,
             bm2_ref, wh2_ref, o_ref):
        ns, nd = _norms(deg_ref[...])
        bh = bh_ref[...]
        h0 = _silu(a0_ref[...] * nd + bh[:, :128])
        h1 = _silu(a1_ref[...] * nd + bh[:, 128:])
        z = _dot(_silu(h0), wm1_ref[:128, :]) + _dot(_silu(h1), wm1_ref[128:, :])
        z = z + bm1_ref[...]
        hm = _dot(_silu(z), wm2_ref[...]) + bm2_ref[...]
        o_ref[...] = _dot(hm * ns, wh2_ref[...])

    return pl.pallas_call(
        body,
        grid=(GRID,),
        in_specs=[
            pl.BlockSpec((BM, 128), _rows),
            pl.BlockSpec((BM, 128), _rows),
            pl.BlockSpec((BM, 2), _rows),
            pl.BlockSpec((1, 256), _full),
            pl.BlockSpec((256, 64), _full),
            pl.BlockSpec((1, 64), _full),
            pl.BlockSpec((64, 256), _full),
            pl.BlockSpec((1, 256), _full),
            pl.BlockSpec((256, 256), _full),
        ],
        out_specs=pl.BlockSpec((BM, 256), _rows),
        out_shape=jax.ShapeDtypeStruct((NPAD, 256), F32),
    )(a0, a1, deg_t, bh1, wm1, bm1, wm2, bm2, wh2)


def _stage_d(a0, a1, deg_t, bh2, w2):
    def body(a0_ref, a1_ref, deg_ref, b_ref, w_ref, o_ref):
        ns, nd = _norms(deg_ref[...])
        b = b_ref[...]
        t0 = _silu(a0_ref[...] * nd + b[:, :128]) * ns
        t1 = _silu(a1_ref[...] * nd + b[:, 128:]) * ns
        o_ref[...] = _dot(t0, w_ref[:128, :]) + _dot(t1, w_ref[128:, :])

    return pl.pallas_call(
        body,
        grid=(GRID,),
        in_specs=[
            pl.BlockSpec((BM, 128), _rows),
            pl.BlockSpec((BM, 128), _rows),
            pl.BlockSpec((BM, 2), _rows),
            pl.BlockSpec((1, 256), _full),
            pl.BlockSpec((256, 128), _full),
        ],
        out_specs=pl.BlockSpec((BM, 128), _rows),
        out_shape=jax.ShapeDtypeStruct((NPAD, 128), F32),
    )(a0, a1, deg_t, bh2, w2)


def _stage_e(a0, a1, deg_t, b2):
    def body(a0_ref, a1_ref, deg_ref, b_ref, o_ref):
        _, nd = _norms(deg_ref[...])
        o_ref[...] = (a0_ref[...] + a1_ref[...]) * nd + b_ref[...]

    return pl.pallas_call(
        body,
        grid=(GRID,),
        in_specs=[
            pl.BlockSpec((BM, 128), _rows),
            pl.BlockSpec((BM, 128), _rows),
            pl.BlockSpec((BM, 2), _rows),
            pl.BlockSpec((1, 128), _full),
        ],
        out_specs=pl.BlockSpec((BM, 128), _rows),
        out_shape=jax.ShapeDtypeStruct((NPAD, 128), F32),
    )(a0, a1, deg_t, b2)


def _split_rows(h):
    n, d = h.shape
    return h.reshape(n, 2, d // 2).reshape(2 * n, d // 2)


def kernel(x, edge_index, W1, b1, Wh1, bh1, Wh2, bh2, Wm1, bm1, Wm2, bm2,
           W2, b2):
    ei = edge_index.astype(jnp.int32)
    eif = ei.reshape(2 * E)
    ein = ei.reshape(2, NS, EPT // CH, CH).transpose(1, 2, 0, 3)
    einl = ei.reshape(2, NC, NS, E // (NC * NS) // CH, CH)
    einl = einl.transpose(1, 2, 3, 0, 4)
    deg_raw = _deg_kernel(eif)
    deg_t = deg_raw[:, :, 0].T

    xp = jnp.pad(x, ((0, NPAD - N), (0, 0)))
    b1r = b1.reshape(1, -1)
    bh1r = bh1.reshape(1, -1)
    bh2r = bh2.reshape(1, -1)
    bm1r = bm1.reshape(1, -1)
    bm2r = bm2.reshape(1, -1)
    b2r = b2.reshape(1, -1)

    xs = _stage_a(xp, deg_t)
    a1 = _agg_last(einl, xs)
    h2 = _stage_b(a1[0], a1[1], deg_t, b1r, W1, Wh1)
    a2 = _agg128(ein, _split_rows(h2))
    h3 = _stage_c(a2[0], a2[1], deg_t, bh1r, Wm1, bm1r, Wm2, bm2r, Wh2)
    a3 = _agg128(ein, _split_rows(h3))
    h4 = _stage_d(a3[0], a3[1], deg_t, bh2r, W2)
    a4 = _agg_last(einl, h4)
    out = _stage_e(a4[0], a4[1], deg_t, b2r)
    return out[:N, :]


# speedup vs baseline: 1.0779x; 1.0114x over previous
import functools

import jax
import jax.numpy as jnp
from jax import lax
from jax.experimental import pallas as pl
from jax.experimental.pallas import tpu as pltpu
from jax.experimental.pallas import tpu_sc as plsc

N = 10000
E = 320000
NC = 2
NS = 16
NPAD = 10240
RPT = NPAD // NS
EPT = E // NS
CH = 80
NCH = EPT // CH

F32 = jnp.float32
BM = 2048
GRID = NPAD // BM


def _sc_mesh():
    return plsc.VectorSubcoreMesh(core_axis_name="c", subcore_axis_name="s")


@functools.partial(
    pl.kernel,
    mesh=_sc_mesh(),
    out_type=jax.ShapeDtypeStruct((NC, NPAD, 128), F32),
    scratch_types=[
        pltpu.VMEM((CH,), jnp.int32),
        pltpu.VMEM((CH,), jnp.int32),
        pltpu.VMEM((CH, 128), F32),
        pltpu.VMEM_SHARED((NPAD, 128), F32),
        pltpu.SemaphoreType.DMA,
        pltpu.SemaphoreType.DMA,
    ],
)
def _deg_kernel(edges, deg_out, idx_a, idx_b, ones_v, acc_s, sem_a, sem_b):
    c = lax.axis_index("c")
    t = lax.axis_index("s")
    idx = (idx_a, idx_b)
    sem = (sem_a, sem_b)

    def fill(val):
        def row(i, carry):
            def lane(j, carry2):
                ones_v[i, pl.ds(j * 16, 16)] = jnp.full((16,), val, F32)
                return carry2

            lax.fori_loop(0, 128 // 16, lane, 0)
            return carry

        lax.fori_loop(0, CH, row, 0)

    fill(0.0)

    def zero_acc(k, carry):
        pltpu.sync_copy(ones_v, acc_s.at[pl.ds(t * RPT + k * CH, CH)])
        return carry

    lax.fori_loop(0, RPT // CH, zero_acc, 0)
    fill(1.0)
    plsc.subcore_barrier()

    def start(ci, b):
        base = c * E + t * EPT + ci * CH
        pltpu.async_copy(edges.at[pl.ds(base, CH)], idx[b], sem[b])

    def finish(b):
        pltpu.make_async_copy(edges.at[pl.ds(0, CH)], idx[b], sem[b]).wait()

    def scatter(b):
        pltpu.sync_copy(ones_v, acc_s.at[idx[b]], add=True)

    start(0, 0)

    def pair(i, carry):
        start(2 * i + 1, 1)
        finish(0)
        scatter(0)
        start((2 * i + 2) % NCH, 0)
        finish(1)
        scatter(1)
        return carry

    lax.fori_loop(0, NCH // 2, pair, 0)
    finish(0)
    plsc.subcore_barrier()

    def out_cp(k, carry):
        r0 = t * RPT + k * CH
        pltpu.sync_copy(acc_s.at[pl.ds(r0, CH)], ones_v)
        pltpu.sync_copy(ones_v, deg_out.at[c, pl.ds(r0, CH)])
        return carry

    lax.fori_loop(0, RPT // CH, out_cp, 0)


def _make_agg(H, edge_split=False):
    ept = E // (NC * NS) if edge_split else EPT
    nch = ept // CH

    @functools.partial(
        pl.kernel,
        mesh=_sc_mesh(),
        out_type=jax.ShapeDtypeStruct((NC, NPAD, H), F32),
        scratch_types=[
            pltpu.VMEM((2, CH), jnp.int32),
            pltpu.VMEM((2, CH), jnp.int32),
            pltpu.VMEM((CH, H), F32),
            pltpu.VMEM((CH, H), F32),
            pltpu.VMEM_SHARED((NPAD, H), F32),
            pltpu.SemaphoreType.DMA,
            pltpu.SemaphoreType.DMA,
            pltpu.SemaphoreType.DMA,
            pltpu.SemaphoreType.DMA,
        ],
    )
    def agg_kernel(edges4, feats, out, idx_a, idx_b, rows_a, rows_b, acc_s,
                   sem_a, sem_b, isem_a, isem_b):
        c = lax.axis_index("c")
        t = lax.axis_index("s")
        idx = (idx_a, idx_b)
        rows = (rows_a, rows_b)
        sem = (sem_a, sem_b)
        isem = (isem_a, isem_b)

        def idx_src(ci):
            if edge_split:
                return edges4.at[c, t, ci]
            return edges4.at[t, ci]

        def load_idx(ci, b):
            pltpu.async_copy(idx_src(ci), idx[b], isem[b])

        load_idx(0, 0)
        load_idx(1, 1)

        def zero_rows(i, carry):
            def zl(j, carry2):
                rows_a[i, pl.ds(j * 16, 16)] = jnp.zeros((16,), F32)
                return carry2

            lax.fori_loop(0, H // 16, zl, 0)
            return carry

        lax.fori_loop(0, CH, zero_rows, 0)

        for k in range(RPT // CH):
            pltpu.async_copy(rows_a, acc_s.at[pl.ds(t * RPT + k * CH, CH)],
                             sem_a)
        for k in range(RPT // CH):
            pltpu.make_async_copy(rows_a,
                                  acc_s.at[pl.ds(t * RPT + k * CH, CH)],
                                  sem_a).wait()
        plsc.subcore_barrier()

        def wait_idx(b):
            pltpu.make_async_copy(idx_src(0), idx[b], isem[b]).wait()

        def launch_gather(b):
            wait_idx(b)
            if not edge_split:
                for j in range(CH // 16):
                    s = idx[b][0, pl.ds(j * 16, 16)]
                    idx[b][0, pl.ds(j * 16, 16)] = s * 2 + c
            pltpu.async_copy(feats.at[idx[b].at[0]], rows[b], sem[b])

        def finish(b):
            pltpu.make_async_copy(feats.at[idx[b].at[0]], rows[b],
                                  sem[b]).wait()

        def scatter(b):
            pltpu.sync_copy(rows[b], acc_s.at[idx[b].at[1]], add=True)

        launch_gather(0)

        def pair(i, carry):
            launch_gather(1)
            finish(0)
            scatter(0)
            load_idx((2 * i + 2) % nch, 0)
            finish(1)
            scatter(1)
            launch_gather(0)
            load_idx((2 * i + 3) % nch, 1)
            return carry

        lax.fori_loop(0, nch // 2, pair, 0)
        finish(0)
        if nch % 2 == 1:
            scatter(0)
        wait_idx(1)
        plsc.subcore_barrier()

        nout = RPT // CH
        for k in range(nout):
            b = k % 2
            r0 = t * RPT + k * CH
            if k >= 2:
                rp = t * RPT + (k - 2) * CH
                pltpu.make_async_copy(rows[b], out.at[c, pl.ds(rp, CH)],
                                      sem[b]).wait()
            pltpu.sync_copy(acc_s.at[pl.ds(r0, CH)], rows[b])
            pltpu.async_copy(rows[b], out.at[c, pl.ds(r0, CH)], sem[b])
        for k in range(nout - 2, nout):
            b = k % 2
            r0 = t * RPT + k * CH
            pltpu.make_async_copy(rows[b], out.at[c, pl.ds(r0, CH)],
                                  sem[b]).wait()

    return agg_kernel


_agg128 = _make_agg(128)

_agg_last = _make_agg(128, edge_split=True)


def _dot(a, b):
    return lax.dot_general(
        a, b, (((1,), (0,)), ((), ())),
        preferred_element_type=F32,
    )


def _silu(v):
    return v / (1.0 + jnp.exp(-v))


def _norms(deg_blk):
    d0 = deg_blk[:, 0:1]
    d1 = deg_blk[:, 1:2]
    ns = jnp.where(d0 > 0, lax.rsqrt(jnp.maximum(d0, 1.0)), 0.0)
    nd = jnp.where(d1 > 0, lax.rsqrt(jnp.maximum(d1, 1.0)), 0.0)
    return ns, nd


def _rows(i):
    return (i, 0)


def _full(i):
    return (0, 0)


def _stage_a(xp, deg_t):
    def body(x_ref, deg_ref, o_ref):
        ns, _ = _norms(deg_ref[...])
        o_ref[...] = x_ref[...] * ns

    return pl.pallas_call(
        body,
        grid=(GRID,),
        in_specs=[
            pl.BlockSpec((BM, 128), _rows),
            pl.BlockSpec((BM, 2), _rows),
        ],
        out_specs=pl.BlockSpec((BM, 128), _rows),
        out_shape=jax.ShapeDtypeStruct((NPAD, 128), F32),
    )(xp, deg_t)


def _stage_b(a0, a1, deg_t, b1, w1, wh1):
    def body(a0_ref, a1_ref, deg_ref, b_ref, w1_ref, w_ref, o_ref):
        ns, nd = _norms(deg_ref[...])
        y = _dot(a0_ref[...] + a1_ref[...], w1_ref[...])
        t = (y * nd + b_ref[...]) * ns
        o_ref[...] = _dot(t, w_ref[...])

    return pl.pallas_call(
        body,
        grid=(GRID,),
        in_specs=[
            pl.BlockSpec((BM, 128), _rows),
            pl.BlockSpec((BM, 128), _rows),
            pl.BlockSpec((BM, 2), _rows),
            pl.BlockSpec((1, 256), _full),
            pl.BlockSpec((128, 256), _full),
            pl.BlockSpec((256, 256), _full),
        ],
        out_specs=pl.BlockSpec((BM, 256), _rows),
        out_shape=jax.ShapeDtypeStruct((NPAD, 256), F32),
    )(a0, a1, deg_t, b1, w1, wh1)


def _stage_c(a0, a1, deg_t, bh1, wm1, bm1, wm2, bm2, wh2):
    def body(a0_ref, a1_ref, deg_ref, bh_ref, wm1_ref, bm1_ref, wm2_ref,
             bm2_ref, wh2_ref, o_ref):
        ns, nd = _norms(deg_ref[...])
        bh = bh_ref[...]
        h0 = _silu(a0_ref[...] * nd + bh[:, :128])
        h1 = _silu(a1_ref[...] * nd + bh[:, 128:])
        z = _dot(_silu(h0), wm1_ref[:128, :]) + _dot(_silu(h1), wm1_ref[128:, :])
        z = z + bm1_ref[...]
        hm = _dot(_silu(z), wm2_ref[...]) + bm2_ref[...]
        o_ref[...] = _dot(hm * ns, wh2_ref[...])

    return pl.pallas_call(
        body,
        grid=(GRID,),
        in_specs=[
            pl.BlockSpec((BM, 128), _rows),
            pl.BlockSpec((BM, 128), _rows),
            pl.BlockSpec((BM, 2), _rows),
            pl.BlockSpec((1, 256), _full),
            pl.BlockSpec((256, 64), _full),
            pl.BlockSpec((1, 64), _full),
            pl.BlockSpec((64, 256), _full),
            pl.BlockSpec((1, 256), _full),
            pl.BlockSpec((256, 256), _full),
        ],
        out_specs=pl.BlockSpec((BM, 256), _rows),
        out_shape=jax.ShapeDtypeStruct((NPAD, 256), F32),
    )(a0, a1, deg_t, bh1, wm1, bm1, wm2, bm2, wh2)


def _stage_d(a0, a1, deg_t, bh2, w2):
    def body(a0_ref, a1_ref, deg_ref, b_ref, w_ref, o_ref):
        ns, nd = _norms(deg_ref[...])
        b = b_ref[...]
        t0 = _silu(a0_ref[...] * nd + b[:, :128]) * ns
        t1 = _silu(a1_ref[...] * nd + b[:, 128:]) * ns
        o_ref[...] = _dot(t0, w_ref[:128, :]) + _dot(t1, w_ref[128:, :])

    return pl.pallas_call(
        body,
        grid=(GRID,),
        in_specs=[
            pl.BlockSpec((BM, 128), _rows),
            pl.BlockSpec((BM, 128), _rows),
            pl.BlockSpec((BM, 2), _rows),
            pl.BlockSpec((1, 256), _full),
            pl.BlockSpec((256, 128), _full),
        ],
        out_specs=pl.BlockSpec((BM, 128), _rows),
        out_shape=jax.ShapeDtypeStruct((NPAD, 128), F32),
    )(a0, a1, deg_t, bh2, w2)


def _stage_e(a0, a1, deg_t, b2):
    def body(a0_ref, a1_ref, deg_ref, b_ref, o_ref):
        _, nd = _norms(deg_ref[...])
        o_ref[...] = (a0_ref[...] + a1_ref[...]) * nd + b_ref[...]

    return pl.pallas_call(
        body,
        grid=(GRID,),
        in_specs=[
            pl.BlockSpec((BM, 128), _rows),
            pl.BlockSpec((BM, 128), _rows),
            pl.BlockSpec((BM, 2), _rows),
            pl.BlockSpec((1, 128), _full),
        ],
        out_specs=pl.BlockSpec((BM, 128), _rows),
        out_shape=jax.ShapeDtypeStruct((NPAD, 128), F32),
    )(a0, a1, deg_t, b2)


def _split_rows(h):
    n, d = h.shape
    return h.reshape(n, 2, d // 2).reshape(2 * n, d // 2)


def kernel(x, edge_index, W1, b1, Wh1, bh1, Wh2, bh2, Wm1, bm1, Wm2, bm2,
           W2, b2):
    ei = edge_index.astype(jnp.int32)
    eif = ei.reshape(2 * E)
    ein = ei.reshape(2, NS, EPT // CH, CH).transpose(1, 2, 0, 3)
    einl = ei.reshape(2, NC, NS, E // (NC * NS) // CH, CH)
    einl = einl.transpose(1, 2, 3, 0, 4)
    deg_raw = _deg_kernel(eif)
    deg_t = deg_raw[:, :, 0].T

    xp = jnp.pad(x, ((0, NPAD - N), (0, 0)))
    b1r = b1.reshape(1, -1)
    bh1r = bh1.reshape(1, -1)
    bh2r = bh2.reshape(1, -1)
    bm1r = bm1.reshape(1, -1)
    bm2r = bm2.reshape(1, -1)
    b2r = b2.reshape(1, -1)

    xs = _stage_a(xp, deg_t)
    a1 = _agg_last(einl, xs)
    h2 = _stage_b(a1[0], a1[1], deg_t, b1r, W1, Wh1)
    a2 = _agg128(ein, _split_rows(h2))
    h3 = _stage_c(a2[0], a2[1], deg_t, bh1r, Wm1, bm1r, Wm2, bm2r, Wh2)
    a3 = _agg128(ein, _split_rows(h3))
    h4 = _stage_d(a3[0], a3[1], deg_t, bh2r, W2)
    a4 = _agg_last(einl, h4)
    out = _stage_e(a4[0], a4[1], deg_t, b2r)
    return out[:N, :]


# speedup vs baseline: 1.0801x; 1.0020x over previous
import functools

import jax
import jax.numpy as jnp
from jax import lax
from jax.experimental import pallas as pl
from jax.experimental.pallas import tpu as pltpu
from jax.experimental.pallas import tpu_sc as plsc

N = 10000
E = 320000
NC = 2
NS = 16
NPAD = 10240
RPT = NPAD // NS
EPT = E // NS
CH = 80
NCH = EPT // CH

F32 = jnp.float32
BM = 2048
GRID = NPAD // BM


def _sc_mesh():
    return plsc.VectorSubcoreMesh(core_axis_name="c", subcore_axis_name="s")


@functools.partial(
    pl.kernel,
    mesh=_sc_mesh(),
    out_type=jax.ShapeDtypeStruct((NC, NPAD, 128), F32),
    scratch_types=[
        pltpu.VMEM((CH,), jnp.int32),
        pltpu.VMEM((CH,), jnp.int32),
        pltpu.VMEM((CH, 128), F32),
        pltpu.VMEM((CH, 128), F32),
        pltpu.VMEM_SHARED((NPAD, 128), F32),
        pltpu.SemaphoreType.DMA,
        pltpu.SemaphoreType.DMA,
    ],
)
def _deg_kernel(edges, deg_out, idx_a, idx_b, ones_v, stg_b, acc_s, sem_a,
                sem_b):
    c = lax.axis_index("c")
    t = lax.axis_index("s")
    idx = (idx_a, idx_b)
    sem = (sem_a, sem_b)

    def fill(val):
        def row(i, carry):
            def lane(j, carry2):
                ones_v[i, pl.ds(j * 16, 16)] = jnp.full((16,), val, F32)
                return carry2

            lax.fori_loop(0, 128 // 16, lane, 0)
            return carry

        lax.fori_loop(0, CH, row, 0)

    fill(0.0)

    for k in range(RPT // CH):
        pltpu.async_copy(ones_v, acc_s.at[pl.ds(t * RPT + k * CH, CH)],
                         sem_a)
    for k in range(RPT // CH):
        pltpu.make_async_copy(ones_v,
                              acc_s.at[pl.ds(t * RPT + k * CH, CH)],
                              sem_a).wait()
    fill(1.0)
    plsc.subcore_barrier()

    def start(ci, b):
        base = c * E + t * EPT + ci * CH
        pltpu.async_copy(edges.at[pl.ds(base, CH)], idx[b], sem[b])

    def finish(b):
        pltpu.make_async_copy(edges.at[pl.ds(0, CH)], idx[b], sem[b]).wait()

    def scatter(b):
        pltpu.sync_copy(ones_v, acc_s.at[idx[b]], add=True)

    start(0, 0)

    def pair(i, carry):
        start(2 * i + 1, 1)
        finish(0)
        scatter(0)
        start((2 * i + 2) % NCH, 0)
        finish(1)
        scatter(1)
        return carry

    lax.fori_loop(0, NCH // 2, pair, 0)
    finish(0)
    plsc.subcore_barrier()

    stg = (ones_v, stg_b)
    nout = RPT // CH
    for k in range(nout):
        b = k % 2
        r0 = t * RPT + k * CH
        if k >= 2:
            rp = t * RPT + (k - 2) * CH
            pltpu.make_async_copy(stg[b], deg_out.at[c, pl.ds(rp, CH)],
                                  sem[b]).wait()
        pltpu.sync_copy(acc_s.at[pl.ds(r0, CH)], stg[b])
        pltpu.async_copy(stg[b], deg_out.at[c, pl.ds(r0, CH)], sem[b])
    for k in range(nout - 2, nout):
        b = k % 2
        r0 = t * RPT + k * CH
        pltpu.make_async_copy(stg[b], deg_out.at[c, pl.ds(r0, CH)],
                              sem[b]).wait()


def _make_agg(H, edge_split=False):
    ept = E // (NC * NS) if edge_split else EPT
    nch = ept // CH

    @functools.partial(
        pl.kernel,
        mesh=_sc_mesh(),
        out_type=jax.ShapeDtypeStruct((NC, NPAD, H), F32),
        scratch_types=[
            pltpu.VMEM((2, CH), jnp.int32),
            pltpu.VMEM((2, CH), jnp.int32),
            pltpu.VMEM((CH, H), F32),
            pltpu.VMEM((CH, H), F32),
            pltpu.VMEM_SHARED((NPAD, H), F32),
            pltpu.SemaphoreType.DMA,
            pltpu.SemaphoreType.DMA,
            pltpu.SemaphoreType.DMA,
            pltpu.SemaphoreType.DMA,
        ],
    )
    def agg_kernel(edges4, feats, out, idx_a, idx_b, rows_a, rows_b, acc_s,
                   sem_a, sem_b, isem_a, isem_b):
        c = lax.axis_index("c")
        t = lax.axis_index("s")
        idx = (idx_a, idx_b)
        rows = (rows_a, rows_b)
        sem = (sem_a, sem_b)
        isem = (isem_a, isem_b)

        def idx_src(ci):
            if edge_split:
                return edges4.at[c, t, ci]
            return edges4.at[t, ci]

        def load_idx(ci, b):
            pltpu.async_copy(idx_src(ci), idx[b], isem[b])

        load_idx(0, 0)
        load_idx(1, 1)

        def zero_rows(i, carry):
            def zl(j, carry2):
                rows_a[i, pl.ds(j * 16, 16)] = jnp.zeros((16,), F32)
                return carry2

            lax.fori_loop(0, H // 16, zl, 0)
            return carry

        lax.fori_loop(0, CH, zero_rows, 0)

        for k in range(RPT // CH):
            pltpu.async_copy(rows_a, acc_s.at[pl.ds(t * RPT + k * CH, CH)],
                             sem_a)
        for k in range(RPT // CH):
            pltpu.make_async_copy(rows_a,
                                  acc_s.at[pl.ds(t * RPT + k * CH, CH)],
                                  sem_a).wait()
        plsc.subcore_barrier()

        def wait_idx(b):
            pltpu.make_async_copy(idx_src(0), idx[b], isem[b]).wait()

        def launch_gather(b):
            wait_idx(b)
            if not edge_split:
                for j in range(CH // 16):
                    s = idx[b][0, pl.ds(j * 16, 16)]
                    idx[b][0, pl.ds(j * 16, 16)] = s * 2 + c
            pltpu.async_copy(feats.at[idx[b].at[0]], rows[b], sem[b])

        def finish(b):
            pltpu.make_async_copy(feats.at[idx[b].at[0]], rows[b],
                                  sem[b]).wait()

        def scatter(b):
            pltpu.sync_copy(rows[b], acc_s.at[idx[b].at[1]], add=True)

        launch_gather(0)

        def pair(i, carry):
            launch_gather(1)
            finish(0)
            scatter(0)
            load_idx((2 * i + 2) % nch, 0)
            finish(1)
            scatter(1)
            launch_gather(0)
            load_idx((2 * i + 3) % nch, 1)
            return carry

        lax.fori_loop(0, nch // 2, pair, 0)
        finish(0)
        if nch % 2 == 1:
            scatter(0)
        wait_idx(1)
        plsc.subcore_barrier()

        nout = RPT // CH
        for k in range(nout):
            b = k % 2
            r0 = t * RPT + k * CH
            if k >= 2:
                rp = t * RPT + (k - 2) * CH
                pltpu.make_async_copy(rows[b], out.at[c, pl.ds(rp, CH)],
                                      sem[b]).wait()
            pltpu.sync_copy(acc_s.at[pl.ds(r0, CH)], rows[b])
            pltpu.async_copy(rows[b], out.at[c, pl.ds(r0, CH)], sem[b])
        for k in range(nout - 2, nout):
            b = k % 2
            r0 = t * RPT + k * CH
            pltpu.make_async_copy(rows[b], out.at[c, pl.ds(r0, CH)],
                                  sem[b]).wait()

    return agg_kernel


_agg128 = _make_agg(128)

_agg_last = _make_agg(128, edge_split=True)


def _dot(a, b):
    return lax.dot_general(
        a, b, (((1,), (0,)), ((), ())),
        preferred_element_type=F32,
    )


def _silu(v):
    return v / (1.0 + jnp.exp(-v))


def _norms(deg_blk):
    d0 = deg_blk[:, 0:1]
    d1 = deg_blk[:, 1:2]
    ns = jnp.where(d0 > 0, lax.rsqrt(jnp.maximum(d0, 1.0)), 0.0)
    nd = jnp.where(d1 > 0, lax.rsqrt(jnp.maximum(d1, 1.0)), 0.0)
    return ns, nd


def _rows(i):
    return (i, 0)


def _full(i):
    return (0, 0)


def _stage_a(xp, deg_t):
    def body(x_ref, deg_ref, o_ref):
        ns, _ = _norms(deg_ref[...])
        o_ref[...] = x_ref[...] * ns

    return pl.pallas_call(
        body,
        grid=(GRID,),
        in_specs=[
            pl.BlockSpec((BM, 128), _rows),
            pl.BlockSpec((BM, 2), _rows),
        ],
        out_specs=pl.BlockSpec((BM, 128), _rows),
        out_shape=jax.ShapeDtypeStruct((NPAD, 128), F32),
    )(xp, deg_t)


def _stage_b(a0, a1, deg_t, b1, w1, wh1):
    def body(a0_ref, a1_ref, deg_ref, b_ref, w1_ref, w_ref, o_ref):
        ns, nd = _norms(deg_ref[...])
        y = _dot(a0_ref[...] + a1_ref[...], w1_ref[...])
        t = (y * nd + b_ref[...]) * ns
        o_ref[...] = _dot(t, w_ref[...])

    return pl.pallas_call(
        body,
        grid=(GRID,),
        in_specs=[
            pl.BlockSpec((BM, 128), _rows),
            pl.BlockSpec((BM, 128), _rows),
            pl.BlockSpec((BM, 2), _rows),
            pl.BlockSpec((1, 256), _full),
            pl.BlockSpec((128, 256), _full),
            pl.BlockSpec((256, 256), _full),
        ],
        out_specs=pl.BlockSpec((BM, 256), _rows),
        out_shape=jax.ShapeDtypeStruct((NPAD, 256), F32),
    )(a0, a1, deg_t, b1, w1, wh1)


def _stage_c(a0, a1, deg_t, bh1, wm1, bm1, wm2, bm2, wh2):
    def body(a0_ref, a1_ref, deg_ref, bh_ref, wm1_ref, bm1_ref, wm2_ref,
             bm2_ref, wh2_ref, o_ref):
        ns, nd = _norms(deg_ref[...])
        bh = bh_ref[...]
        h0 = _silu(a0_ref[...] * nd + bh[:, :128])
        h1 = _silu(a1_ref[...] * nd + bh[:, 128:])
        z = _dot(_silu(h0), wm1_ref[:128, :]) + _dot(_silu(h1), wm1_ref[128:, :])
        z = z + bm1_ref[...]
        hm = _dot(_silu(z), wm2_ref[...]) + bm2_ref[...]
        o_ref[...] = _dot(hm * ns, wh2_ref[...])

    return pl.pallas_call(
        body,
        grid=(GRID,),
        in_specs=[
            pl.BlockSpec((BM, 128), _rows),
            pl.BlockSpec((BM, 128), _rows),
            pl.BlockSpec((BM, 2), _rows),
            pl.BlockSpec((1, 256), _full),
            pl.BlockSpec((256, 64), _full),
            pl.BlockSpec((1, 64), _full),
            pl.BlockSpec((64, 256), _full),
            pl.BlockSpec((1, 256), _full),
            pl.BlockSpec((256, 256), _full),
        ],
        out_specs=pl.BlockSpec((BM, 256), _rows),
        out_shape=jax.ShapeDtypeStruct((NPAD, 256), F32),
    )(a0, a1, deg_t, bh1, wm1, bm1, wm2, bm2, wh2)


def _stage_d(a0, a1, deg_t, bh2, w2):
    def body(a0_ref, a1_ref, deg_ref, b_ref, w_ref, o_ref):
        ns, nd = _norms(deg_ref[...])
        b = b_ref[...]
        t0 = _silu(a0_ref[...] * nd + b[:, :128]) * ns
        t1 = _silu(a1_ref[...] * nd + b[:, 128:]) * ns
        o_ref[...] = _dot(t0, w_ref[:128, :]) + _dot(t1, w_ref[128:, :])

    return pl.pallas_call(
        body,
        grid=(GRID,),
        in_specs=[
            pl.BlockSpec((BM, 128), _rows),
            pl.BlockSpec((BM, 128), _rows),
            pl.BlockSpec((BM, 2), _rows),
            pl.BlockSpec((1, 256), _full),
            pl.BlockSpec((256, 128), _full),
        ],
        out_specs=pl.BlockSpec((BM, 128), _rows),
        out_shape=jax.ShapeDtypeStruct((NPAD, 128), F32),
    )(a0, a1, deg_t, bh2, w2)


def _stage_e(a0, a1, deg_t, b2):
    def body(a0_ref, a1_ref, deg_ref, b_ref, o_ref):
        _, nd = _norms(deg_ref[...])
        o_ref[...] = (a0_ref[...] + a1_ref[...]) * nd + b_ref[...]

    return pl.pallas_call(
        body,
        grid=(GRID,),
        in_specs=[
            pl.BlockSpec((BM, 128), _rows),
            pl.BlockSpec((BM, 128), _rows),
            pl.BlockSpec((BM, 2), _rows),
            pl.BlockSpec((1, 128), _full),
        ],
        out_specs=pl.BlockSpec((BM, 128), _rows),
        out_shape=jax.ShapeDtypeStruct((NPAD, 128), F32),
    )(a0, a1, deg_t, b2)


def _split_rows(h):
    n, d = h.shape
    return h.reshape(n, 2, d // 2).reshape(2 * n, d // 2)


def kernel(x, edge_index, W1, b1, Wh1, bh1, Wh2, bh2, Wm1, bm1, Wm2, bm2,
           W2, b2):
    ei = edge_index.astype(jnp.int32)
    eif = ei.reshape(2 * E)
    ein = ei.reshape(2, NS, EPT // CH, CH).transpose(1, 2, 0, 3)
    einl = ei.reshape(2, NC, NS, E // (NC * NS) // CH, CH)
    einl = einl.transpose(1, 2, 3, 0, 4)
    deg_raw = _deg_kernel(eif)
    deg_t = deg_raw[:, :, 0].T

    xp = jnp.pad(x, ((0, NPAD - N), (0, 0)))
    b1r = b1.reshape(1, -1)
    bh1r = bh1.reshape(1, -1)
    bh2r = bh2.reshape(1, -1)
    bm1r = bm1.reshape(1, -1)
    bm2r = bm2.reshape(1, -1)
    b2r = b2.reshape(1, -1)

    xs = _stage_a(xp, deg_t)
    a1 = _agg_last(einl, xs)
    h2 = _stage_b(a1[0], a1[1], deg_t, b1r, W1, Wh1)
    a2 = _agg128(ein, _split_rows(h2))
    h3 = _stage_c(a2[0], a2[1], deg_t, bh1r, Wm1, bm1r, Wm2, bm2r, Wh2)
    a3 = _agg128(ein, _split_rows(h3))
    h4 = _stage_d(a3[0], a3[1], deg_t, bh2r, W2)
    a4 = _agg_last(einl, h4)
    out = _stage_e(a4[0], a4[1], deg_t, b2r)
    return out[:N, :]
